# baseline (speedup 1.0000x reference)
import functools

import jax
import jax.numpy as jnp
from jax.experimental import pallas as pl
from jax.experimental.pallas import tpu as pltpu

_VMEM = pl.BlockSpec(memory_space=pltpu.MemorySpace.VMEM)


def _rup(x, m):
    return (x + m - 1) // m * m


def _conv_kernel(xt_ref, w_ref, s_ref, b_ref, o_ref, cols_ref, *, th, Wc, Wo,
                 C, act):
    for di in range(3):
        for dj in range(3):
            t = di * 3 + dj
            cols_ref[:, t * C:(t + 1) * C] = (
                xt_ref[0, 0, di:di + th, dj:dj + Wc, :].reshape(th * Wc, C))
    y = jnp.dot(cols_ref[...], w_ref[...], preferred_element_type=jnp.float32)
    y = y * s_ref[...] + b_ref[...]
    if act == "relu":
        y = jnp.maximum(y, 0.0)
    y = y.reshape(th, Wc, -1)
    if Wo < Wc:
        mask = jax.lax.broadcasted_iota(jnp.int32, (th, Wc, 1), 1) < Wo
        y = jnp.where(mask, y, 0.0)
    o_ref[...] = y[None, None].astype(o_ref.dtype)


def _conv3x3_same(x, w, scale, shift, Wo, th=None, act="relu"):
    N, H, Win, C = x.shape
    Cout = w.shape[1]
    Wc = _rup(Wo, 8)
    if th is None:
        th = H
    nh = -(-H // th)
    Hp = nh * th
    xp = jnp.pad(x.astype(jnp.bfloat16),
                 ((0, 0), (1, Hp - H + 1), (1, Wc + 1 - Win), (0, 0)))
    xt = jnp.stack([xp[:, i * th:i * th + th + 2] for i in range(nh)], axis=1)
    out = pl.pallas_call(
        functools.partial(_conv_kernel, th=th, Wc=Wc, Wo=Wo, C=C, act=act),
        out_shape=jax.ShapeDtypeStruct((N, nh, th, Wc, Cout), jnp.bfloat16),
        grid=(N, nh),
        in_specs=[
            pl.BlockSpec((1, 1, th + 2, Wc + 2, C),
                         lambda n, i: (n, i, 0, 0, 0)),
            pl.BlockSpec((9 * C, Cout), lambda n, i: (0, 0)),
            pl.BlockSpec((1, Cout), lambda n, i: (0, 0)),
            pl.BlockSpec((1, Cout), lambda n, i: (0, 0)),
        ],
        out_specs=pl.BlockSpec((1, 1, th, Wc, Cout),
                               lambda n, i: (n, i, 0, 0, 0)),
        scratch_shapes=[pltpu.VMEM((th * Wc, 9 * C), jnp.bfloat16)],
        compiler_params=pltpu.CompilerParams(
            dimension_semantics=("parallel", "parallel")),
    )(xt, w.astype(jnp.bfloat16), scale.astype(jnp.float32).reshape(1, Cout),
      shift.astype(jnp.float32).reshape(1, Cout))
    out = out.reshape(N, Hp, Wc, Cout)
    return out[:, :H] if Hp > H else out


def _maxpool_3x3_s2(x, H, W):
    N, C = x.shape[0], x.shape[3]
    Ho, Wo = (H - 3) // 2 + 1, (W - 3) // 2 + 1
    out = None
    for di in range(3):
        for dj in range(3):
            v = jax.lax.slice(x[:, :H], (0, di, dj, 0),
                              (N, di + 2 * Ho - 1, dj + 2 * Wo - 1, C),
                              (1, 2, 2, 1))
            out = v if out is None else jnp.maximum(out, v)
    return out


def _sconv_kernel(a_ref, w_ref, s_ref, b_ref, o_ref):
    y = jnp.dot(a_ref[...], w_ref[...], preferred_element_type=jnp.float32)
    y = jnp.maximum(y * s_ref[...] + b_ref[...], 0.0)
    o_ref[...] = y.astype(o_ref.dtype)


def _gru_kernel(a_ref, w_ref, b_ref, o_ref, *, C):
    y = jnp.dot(a_ref[...], w_ref[...], preferred_element_type=jnp.float32)
    y = y + b_ref[...]
    u = jax.nn.sigmoid(y[:, :C]).astype(jnp.bfloat16).astype(jnp.float32)
    c = jnp.tanh(y[:, C:]).astype(jnp.bfloat16).astype(jnp.float32)
    o_ref[...] = (u * c).astype(jnp.bfloat16)


def _head_kernel(a_ref, w1_ref, b1_ref, w2_ref, b2_ref, o_ref):
    y = jnp.dot(a_ref[...], w1_ref[...], preferred_element_type=jnp.float32)
    y = jnp.maximum(y + b1_ref[...], 0.0).astype(jnp.bfloat16)
    o_ref[...] = (jnp.dot(y, w2_ref[...], preferred_element_type=jnp.float32)
                  + b2_ref[...])


def _full_call(kern, out_shape, *args):
    return pl.pallas_call(
        kern, out_shape=out_shape,
        in_specs=[_VMEM] * len(args), out_specs=_VMEM)(*args)


def _im2col_same_xla(h):
    N, H, W, C = h.shape
    hp = jnp.pad(h, ((0, 0), (1, 1), (1, 1), (0, 0)))
    cols = jnp.concatenate(
        [hp[:, di:di + H, dj:dj + W, :] for di in range(3) for dj in range(3)],
        axis=-1)
    return cols.reshape(N * H * W, 9 * C)


def _im2col_valid_s2_xla(h):
    N, H, W, C = h.shape
    Ho, Wo = (H - 3) // 2 + 1, (W - 3) // 2 + 1
    cols = jnp.concatenate(
        [h[:, di::2, dj::2, :][:, :Ho, :Wo, :]
         for di in range(3) for dj in range(3)], axis=-1)
    return cols.reshape(N * Ho * Wo, 9 * C), Ho, Wo


_CFG = [16, 16, "MP", 32, 32, "MP", 64, 64, "MP", 128, 128, "MP", 256, 256]


def kernel(conv_w_0, conv_w_1, conv_w_2, conv_w_3, conv_w_4, conv_w_5,
           conv_w_6, conv_w_7, conv_w_8, conv_w_9, conv_scale_0, conv_scale_1,
           conv_scale_2, conv_scale_3, conv_scale_4, conv_scale_5,
           conv_scale_6, conv_scale_7, conv_scale_8, conv_scale_9,
           conv_shift_0, conv_shift_1, conv_shift_2, conv_shift_3,
           conv_shift_4, conv_shift_5, conv_shift_6, conv_shift_7,
           conv_shift_8, conv_shift_9, sconv_w, sconv_scale, sconv_shift,
           gru_w_ur, gru_b_ur, gru_w_out, gru_b_out, lin_w1, lin_b1, lin_w2,
           lin_b2, x):
    conv_w = [conv_w_0, conv_w_1, conv_w_2, conv_w_3, conv_w_4, conv_w_5,
              conv_w_6, conv_w_7, conv_w_8, conv_w_9]
    conv_scale = [conv_scale_0, conv_scale_1, conv_scale_2, conv_scale_3,
                  conv_scale_4, conv_scale_5, conv_scale_6, conv_scale_7,
                  conv_scale_8, conv_scale_9]
    conv_shift = [conv_shift_0, conv_shift_1, conv_shift_2, conv_shift_3,
                  conv_shift_4, conv_shift_5, conv_shift_6, conv_shift_7,
                  conv_shift_8, conv_shift_9]

    N = x.shape[0]
    h = x.astype(jnp.bfloat16)
    H, W = 191, 255
    tile = {0: 24, 1: 48, 2: 95, 3: 95}
    ci = 0
    for item in _CFG:
        if item == "MP":
            h = _maxpool_3x3_s2(h, H, W)
            H, W = (H - 3) // 2 + 1, (W - 3) // 2 + 1
        else:
            h = _conv3x3_same(h, conv_w[ci], conv_scale[ci], conv_shift[ci],
                              Wo=W, th=tile.get(ci))
            ci += 1

    cols_s, Ho, Wo = _im2col_valid_s2_xla(h[:, :, :W, :])
    C = sconv_w.shape[1]
    hs = _full_call(
        _sconv_kernel, jax.ShapeDtypeStruct((cols_s.shape[0], C), jnp.bfloat16),
        cols_s, sconv_w, sconv_scale.reshape(1, C).astype(jnp.float32),
        sconv_shift.reshape(1, C).astype(jnp.float32))
    hs = hs.reshape(N, Ho, Wo, C)

    w_u = gru_w_ur.reshape(9, 2 * C, 2 * C)[:, :C, :C]
    w_c = gru_w_out.reshape(9, 2 * C, C)[:, :C, :]
    w_uc = jnp.concatenate([w_u, w_c], axis=-1).reshape(9 * C, 2 * C)
    b_uc = jnp.concatenate([gru_b_ur[:C], gru_b_out]).reshape(1, 2 * C)
    cols_g = _im2col_same_xla(hs)
    ns = _full_call(
        functools.partial(_gru_kernel, C=C),
        jax.ShapeDtypeStruct((cols_g.shape[0], C), jnp.bfloat16),
        cols_g, w_uc.astype(jnp.bfloat16), b_uc.astype(jnp.float32))
    new_state = ns.reshape(N, Ho, Wo, C)

    sp = Ho * Wo
    w1p = lin_w1.reshape(C, sp, -1).transpose(1, 0, 2).reshape(sp * C, -1)
    flat = ns.reshape(N, sp * C)
    out = _full_call(
        _head_kernel,
        jax.ShapeDtypeStruct((N, lin_w2.shape[1]), jnp.float32),
        flat, w1p.astype(jnp.bfloat16),
        lin_b1.reshape(1, -1).astype(jnp.float32), lin_w2.astype(jnp.bfloat16),
        lin_b2.reshape(1, -1).astype(jnp.float32))
    return out.reshape(N, Ho, Wo, -1), new_state
